# baseline (speedup 1.0000x reference)
import functools

import jax
import jax.numpy as jnp
from jax import lax
from jax.experimental import pallas as pl
from jax.experimental.pallas import tpu as pltpu
from jax.experimental.pallas import tpu_sc as plsc

N = 10000
E = 160000
D = 256
ED = 16
H = 2 * D

NPAD = 10240
NC = 2
NS = 16
CHUNK = 80
SPAN = E // NS
NCHUNK = SPAN // CHUNK
ROWS_PER_TILE = NPAD // NS
CH = CHUNK
NCH = NCHUNK
RPT = ROWS_PER_TILE
CROWS0 = 63


def _ea_stats_body(ea_ref, s_ref, ss_ref):
    @pl.when(pl.program_id(0) == 0)
    def _():
        s_ref[...] = jnp.zeros_like(s_ref)
        ss_ref[...] = jnp.zeros_like(ss_ref)
    ea = ea_ref[...]
    s_ref[...] += jnp.sum(ea, axis=0, keepdims=True)
    ss_ref[...] += jnp.sum(ea * ea, axis=0, keepdims=True)


def _ea_stats(edge_attr):
    be = 8000
    return pl.pallas_call(
        _ea_stats_body,
        grid=(E // be,),
        in_specs=[pl.BlockSpec((be, ED), lambda i: (i, 0))],
        out_specs=[pl.BlockSpec((1, ED), lambda i: (0, 0))] * 2,
        out_shape=[jax.ShapeDtypeStruct((1, ED), jnp.float32)] * 2,
        compiler_params=pltpu.CompilerParams(
            dimension_semantics=("arbitrary",)),
    )(edge_attr)


def _h_stats_body(ea_ref, w1_ref, b1_ref, s_ref, ss_ref):
    @pl.when(pl.program_id(0) == 0)
    def _():
        s_ref[...] = jnp.zeros_like(s_ref)
        ss_ref[...] = jnp.zeros_like(ss_ref)
    h = jnp.dot(ea_ref[...], w1_ref[...],
                preferred_element_type=jnp.float32) + b1_ref[...]
    h = jnp.maximum(h, 0.0)
    s_ref[...] += jnp.sum(h, axis=0, keepdims=True)
    ss_ref[...] += jnp.sum(h * h, axis=0, keepdims=True)


def _h_stats(edge_attr, w1f, b1f):
    be = 2000
    return pl.pallas_call(
        _h_stats_body,
        grid=(E // be,),
        in_specs=[
            pl.BlockSpec((be, ED), lambda i: (i, 0)),
            pl.BlockSpec((ED, H), lambda i: (0, 0)),
            pl.BlockSpec((1, H), lambda i: (0, 0)),
        ],
        out_specs=[pl.BlockSpec((1, H), lambda i: (0, 0))] * 2,
        out_shape=[jax.ShapeDtypeStruct((1, H), jnp.float32)] * 2,
        compiler_params=pltpu.CompilerParams(
            dimension_semantics=("arbitrary",)),
    )(edge_attr, w1f, b1f)


def _edge_w_body(ea_ref, w1_ref, b1_ref, w2_ref, b2_ref, lo_ref, hi_ref):
    h = jnp.dot(ea_ref[...], w1_ref[...],
                preferred_element_type=jnp.float32) + b1_ref[...]
    h = jnp.maximum(h, 0.0)
    w = jnp.dot(h, w2_ref[...],
                preferred_element_type=jnp.float32) + b2_ref[...]
    lo_ref[...] = w[:, :128]
    hi_ref[...] = w[:, 128:]


def _edge_w(edge_attr, w1f, b1f, w2f, b2f):
    be = 2000
    return pl.pallas_call(
        _edge_w_body,
        grid=(E // be,),
        in_specs=[
            pl.BlockSpec((be, ED), lambda i: (i, 0)),
            pl.BlockSpec((ED, H), lambda i: (0, 0)),
            pl.BlockSpec((1, H), lambda i: (0, 0)),
            pl.BlockSpec((H, D), lambda i: (0, 0)),
            pl.BlockSpec((1, D), lambda i: (0, 0)),
        ],
        out_specs=[pl.BlockSpec((be, 128), lambda i: (i, 0))] * 2,
        out_shape=[jax.ShapeDtypeStruct((E, 128), jnp.float32)] * 2,
        compiler_params=pltpu.CompilerParams(
            dimension_semantics=("arbitrary",)),
    )(edge_attr, w1f, b1f, w2f, b2f)


def _node1_body(alo_ref, ahi_ref, ca_ref, cb_ref, bias_ref,
                m0_ref, s_ref, ss_ref):
    @pl.when(pl.program_id(0) == 0)
    def _():
        s_ref[...] = jnp.zeros_like(s_ref)
        ss_ref[...] = jnp.zeros_like(ss_ref)
    agg = jnp.concatenate([alo_ref[...], ahi_ref[...]], axis=1)
    cnt = jnp.maximum(ca_ref[:, 0:1] + cb_ref[:, 0:1], 1.0)
    m0 = jnp.maximum(agg / cnt + bias_ref[...], 0.0)
    m0_ref[...] = m0
    s_ref[...] += jnp.sum(m0, axis=0, keepdims=True)
    ss_ref[...] += jnp.sum(m0 * m0, axis=0, keepdims=True)


def _node1(agg_lo, agg_hi, cnt_a, cnt_b, conv_bias):
    bn = 2000
    return pl.pallas_call(
        _node1_body,
        grid=(N // bn,),
        in_specs=[
            pl.BlockSpec((bn, 128), lambda i: (i, 0)),
            pl.BlockSpec((bn, 128), lambda i: (i, 0)),
            pl.BlockSpec((bn, 128), lambda i: (i, 0)),
            pl.BlockSpec((bn, 128), lambda i: (i, 0)),
            pl.BlockSpec((1, D), lambda i: (0, 0)),
        ],
        out_specs=[
            pl.BlockSpec((bn, D), lambda i: (i, 0)),
            pl.BlockSpec((1, D), lambda i: (0, 0)),
            pl.BlockSpec((1, D), lambda i: (0, 0)),
        ],
        out_shape=[
            jax.ShapeDtypeStruct((N, D), jnp.float32),
            jax.ShapeDtypeStruct((1, D), jnp.float32),
            jax.ShapeDtypeStruct((1, D), jnp.float32),
        ],
        compiler_params=pltpu.CompilerParams(
            dimension_semantics=("arbitrary",)),
    )(agg_lo, agg_hi, cnt_a, cnt_b, conv_bias)


def _node2_body(m0_ref, w3_ref, b3_ref, t_ref, s_ref, ss_ref):
    @pl.when(pl.program_id(0) == 0)
    def _():
        s_ref[...] = jnp.zeros_like(s_ref)
        ss_ref[...] = jnp.zeros_like(ss_ref)
    t = jnp.dot(m0_ref[...], w3_ref[...],
                preferred_element_type=jnp.float32) + b3_ref[...]
    t = jnp.maximum(t, 0.0)
    t_ref[...] = t
    s_ref[...] += jnp.sum(t, axis=0, keepdims=True)
    ss_ref[...] += jnp.sum(t * t, axis=0, keepdims=True)


def _node2(m0, w3f, b3f):
    bn = 2000
    return pl.pallas_call(
        _node2_body,
        grid=(N // bn,),
        in_specs=[
            pl.BlockSpec((bn, D), lambda i: (i, 0)),
            pl.BlockSpec((D, H), lambda i: (0, 0)),
            pl.BlockSpec((1, H), lambda i: (0, 0)),
        ],
        out_specs=[
            pl.BlockSpec((bn, H), lambda i: (i, 0)),
            pl.BlockSpec((1, H), lambda i: (0, 0)),
            pl.BlockSpec((1, H), lambda i: (0, 0)),
        ],
        out_shape=[
            jax.ShapeDtypeStruct((N, H), jnp.float32),
            jax.ShapeDtypeStruct((1, H), jnp.float32),
            jax.ShapeDtypeStruct((1, H), jnp.float32),
        ],
        compiler_params=pltpu.CompilerParams(
            dimension_semantics=("arbitrary",)),
    )(m0, w3f, b3f)


def _node3_body(x_ref, t_ref, w4_ref, b4_ref, o_ref):
    o_ref[...] = x_ref[...] + jnp.dot(
        t_ref[...], w4_ref[...],
        preferred_element_type=jnp.float32) + b4_ref[...]


def _node3(x, t, w4f, b4f):
    bn = 2000
    return pl.pallas_call(
        _node3_body,
        grid=(N // bn,),
        in_specs=[
            pl.BlockSpec((bn, D), lambda i: (i, 0)),
            pl.BlockSpec((bn, H), lambda i: (i, 0)),
            pl.BlockSpec((H, D), lambda i: (0, 0)),
            pl.BlockSpec((1, D), lambda i: (0, 0)),
        ],
        out_specs=pl.BlockSpec((bn, D), lambda i: (i, 0)),
        out_shape=jax.ShapeDtypeStruct((N, D), jnp.float32),
        compiler_params=pltpu.CompilerParams(
            dimension_semantics=("arbitrary",)),
    )(x, t, w4f, b4f)


def _sc_body(xlo, xhi, wlo, whi, src_hbm, dst_hbm,
         agg_lo, agg_hi,
         src_a, src_b, dst_a, dst_b, w_a, w_b, xg_a, xg_b, agg_sh,
         sem_wa, sem_wb, sem_ga, sem_gb, sem_sa, sem_sb):
    c = lax.axis_index("c")
    s = lax.axis_index("s")

    SRC = {0: src_a, 1: src_b}
    DST = {0: dst_a, 1: dst_b}
    W = {0: w_a, 1: w_b}
    XG = {0: xg_a, 1: xg_b}
    SW = {0: sem_wa, 1: sem_wb}
    SG = {0: sem_ga, 1: sem_gb}
    SS = {0: sem_sa, 1: sem_sb}

    def zero_vmem(ref, rows):
        z = jnp.zeros((16,), jnp.float32)

        def row(b, _):
            for j in range(8):
                ref[b, pl.ds(j * 16, 16)] = z
            return 0
        lax.fori_loop(0, rows, row, 0)

    def zero_stripe():
        zero_vmem(xg_a, CH)
        for r in range(RPT // CH):
            pltpu.sync_copy(
                xg_a, agg_sh.at[pl.ds(s * RPT + r * CH, CH)])

    def half(x_hbm, w_hbm, agg_hbm):
        zero_stripe()
        plsc.subcore_barrier()

        def issue_loads(cid, q):
            base = s * SPAN + cid * CH
            pltpu.async_copy(src_hbm.at[pl.ds(base, CH)], SRC[q], SW[q])
            pltpu.async_copy(dst_hbm.at[pl.ds(base, CH)], DST[q], SW[q])
            pltpu.async_copy(w_hbm.at[pl.ds(base, CH)], W[q], SW[q])

        def wait_loads(q):
            pltpu.make_async_copy(src_hbm.at[pl.ds(0, CH)], SRC[q],
                                  SW[q]).wait()
            pltpu.make_async_copy(dst_hbm.at[pl.ds(0, CH)], DST[q],
                                  SW[q]).wait()
            pltpu.make_async_copy(w_hbm.at[pl.ds(0, CH)], W[q], SW[q]).wait()

        def wait_g(q):
            pltpu.make_async_copy(x_hbm.at[pl.ds(0, CH)], XG[q],
                                  SG[q]).wait()

        def wait_s(q):
            pltpu.make_async_copy(w_hbm.at[pl.ds(0, CH)], W[q], SS[q]).wait()

        def mul(q):
            wq, xq = W[q], XG[q]

            def mul_row(b, _):
                for j in range(8):
                    sl = pl.ds(j * 16, 16)
                    wq[b, sl] = wq[b, sl] * xq[b, sl]
                return 0
            lax.fori_loop(0, CH, mul_row, 0)

        def sub(cid, q, has_prev, has_next):
            p = 1 - q
            wait_loads(q)
            pltpu.async_copy(x_hbm.at[SRC[q]], XG[q], SG[q])
            if has_prev is True:
                wait_s(p)
            else:
                @pl.when(has_prev)
                def _():
                    wait_s(p)
            if has_next:
                issue_loads(cid + 1, p)
            wait_g(q)
            mul(q)
            pltpu.async_copy(W[q], agg_sh.at[DST[q]], SS[q], add=True)

        issue_loads(0, 0)

        def pair(g, _):
            sub(2 * g, 0, g > 0, True)
            sub(2 * g + 1, 1, True, True)
            return 0

        lax.fori_loop(0, NCH // 2, pair, 0)
        sub(NCH - 1, 0, True, False)
        wait_s(0)
        plsc.subcore_barrier()
        row0 = s * RPT
        pltpu.sync_copy(agg_sh.at[pl.ds(row0, RPT)],
                        agg_hbm.at[pl.ds(row0, RPT)])

    @pl.when(c == 0)
    def _():
        half(xlo, wlo, agg_lo)

    @pl.when(c == 1)
    def _():
        half(xhi, whi, agg_hi)


def _sc_cnt_body(dst_hbm, cnt_a, cnt_b, dst_a, dst_b, val_v, agg_sh,
                 sem_wa, sem_wb, sem_sa, sem_sb):
    c = lax.axis_index("c")
    s = lax.axis_index("s")

    DST = {0: dst_a, 1: dst_b}
    SW = {0: sem_wa, 1: sem_wb}
    SS = {0: sem_sa, 1: sem_sb}

    def zero_val():
        z = jnp.zeros((16,), jnp.float32)

        def row(b, _):
            for j in range(8):
                val_v[b, pl.ds(j * 16, 16)] = z
            return 0
        lax.fori_loop(0, CH, row, 0)

    def half(cnt_hbm, jlo, jhi):
        zero_val()
        for r in range(RPT // CH):
            pltpu.sync_copy(
                val_v, agg_sh.at[pl.ds(s * RPT + r * CH, CH)])
        e0 = jnp.where(lax.iota(jnp.int32, 16) == 0, 1.0, 0.0)

        def srow(b, _):
            val_v[b, pl.ds(0, 16)] = e0
            return 0
        lax.fori_loop(0, CH, srow, 0)
        plsc.subcore_barrier()

        def cissue(j, q):
            base = s * SPAN + j * CH
            pltpu.async_copy(dst_hbm.at[pl.ds(base, CH)], DST[q], SW[q])

        def cwait(q):
            pltpu.make_async_copy(dst_hbm.at[pl.ds(0, CH)], DST[q],
                                  SW[q]).wait()

        def wait_s(q):
            pltpu.make_async_copy(cnt_hbm.at[pl.ds(0, CH)], val_v,
                                  SS[q]).wait()

        def csub(j, q, has_prev, has_next):
            p = 1 - q
            cwait(q)
            if has_prev is True:
                wait_s(p)
            elif has_prev is not False:
                @pl.when(has_prev)
                def _():
                    wait_s(p)
            if has_next:
                cissue(j + 1, p)
            pltpu.async_copy(val_v, agg_sh.at[DST[q]], SS[q], add=True)

        ncnt = jhi - jlo
        npair = ncnt // 2
        odd = ncnt % 2 == 1
        cissue(jlo, 0)

        def cpair(g, _):
            j = jlo + 2 * g
            csub(j, 0, g > 0, True)
            csub(j + 1, 1, True, True)
            return 0

        lax.fori_loop(0, npair - 1, cpair, 0)
        j = jlo + 2 * (npair - 1)
        csub(j, 0, npair > 1, True)
        csub(j + 1, 1, True, odd)
        if odd:
            csub(jhi - 1, 0, True, False)
            wait_s(0)
        else:
            wait_s(1)
        plsc.subcore_barrier()
        row0 = s * RPT
        pltpu.sync_copy(agg_sh.at[pl.ds(row0, RPT)],
                        cnt_hbm.at[pl.ds(row0, RPT)])

    @pl.when(c == 0)
    def _():
        half(cnt_a, 0, CROWS0)

    @pl.when(c == 1)
    def _():
        half(cnt_b, CROWS0, NCH)


@functools.cache
def _sc_counts():
    return pl.kernel(
        _sc_cnt_body,
        out_type=[jax.ShapeDtypeStruct((NPAD, 128), jnp.float32)] * 2,
        mesh=plsc.VectorSubcoreMesh(
            core_axis_name="c", subcore_axis_name="s",
            num_cores=NC, num_subcores=NS),
        scratch_types=[pltpu.VMEM((CH,), jnp.int32),
                       pltpu.VMEM((CH,), jnp.int32),
                       pltpu.VMEM((CH, 128), jnp.float32),
                       pltpu.VMEM_SHARED((NPAD, 128), jnp.float32),
                       pltpu.SemaphoreType.DMA,
                       pltpu.SemaphoreType.DMA,
                       pltpu.SemaphoreType.DMA,
                       pltpu.SemaphoreType.DMA])


@functools.cache
def _sc_aggregate():
    return pl.kernel(
        _sc_body,
        out_type=[jax.ShapeDtypeStruct((NPAD, 128), jnp.float32)] * 2,
        mesh=plsc.VectorSubcoreMesh(
            core_axis_name="c", subcore_axis_name="s",
            num_cores=NC, num_subcores=NS),
        scratch_types=[pltpu.VMEM((CH,), jnp.int32),
                       pltpu.VMEM((CH,), jnp.int32),
                       pltpu.VMEM((CH,), jnp.int32),
                       pltpu.VMEM((CH,), jnp.int32),
                       pltpu.VMEM((CH, 128), jnp.float32),
                       pltpu.VMEM((CH, 128), jnp.float32),
                       pltpu.VMEM((CH, 128), jnp.float32),
                       pltpu.VMEM((CH, 128), jnp.float32),
                       pltpu.VMEM_SHARED((NPAD, 128), jnp.float32),
                       pltpu.SemaphoreType.DMA,
                       pltpu.SemaphoreType.DMA,
                       pltpu.SemaphoreType.DMA,
                       pltpu.SemaphoreType.DMA,
                       pltpu.SemaphoreType.DMA,
                       pltpu.SemaphoreType.DMA])


def kernel(x, edge_index, edge_attr, g1, b1, W1, bw1, g2, b2, W2, bw2,
           conv_bias, g3, b3, W3, bw3, g4, b4, W4, bw4):
    src = edge_index[0]
    dst = edge_index[1]
    x_lo = x[:, :128]
    x_hi = x[:, 128:]

    cnt_a, cnt_b = _sc_counts()(dst)

    s1, ss1 = _ea_stats(edge_attr)
    mu1 = s1[0] / E
    var1 = ss1[0] / E - mu1 * mu1
    sc1 = g1 / jnp.sqrt(var1 + 1e-5)
    w1f = W1 * sc1[:, None]
    b1f = (bw1 + (b1 - mu1 * sc1) @ W1)[None, :]

    s2, ss2 = _h_stats(edge_attr, w1f, b1f)
    mu2 = s2[0] / E
    var2 = ss2[0] / E - mu2 * mu2
    sc2 = g2 / jnp.sqrt(var2 + 1e-5)
    w2f = W2 * sc2[:, None]
    b2f = (bw2 + (b2 - mu2 * sc2) @ W2)[None, :]

    w_lo, w_hi = _edge_w(edge_attr, w1f, b1f, w2f, b2f)

    agg_lo, agg_hi = _sc_aggregate()(x_lo, x_hi, w_lo, w_hi, src, dst)

    m0, s3, ss3 = _node1(agg_lo, agg_hi, cnt_a, cnt_b, conv_bias[None, :])
    mu3 = s3[0] / N
    var3 = ss3[0] / N - mu3 * mu3
    sc3 = g3 / jnp.sqrt(var3 + 1e-5)
    w3f = W3 * sc3[:, None]
    b3f = (bw3 + (b3 - mu3 * sc3) @ W3)[None, :]

    t, s4, ss4 = _node2(m0, w3f, b3f)
    mu4 = s4[0] / N
    var4 = ss4[0] / N - mu4 * mu4
    sc4 = g4 / jnp.sqrt(var4 + 1e-5)
    w4f = W4 * sc4[:, None]
    b4f = (bw4 + (b4 - mu4 * sc4) @ W4)[None, :]

    return _node3(x, t, w4f, b4f)


# speedup vs baseline: 1.0042x; 1.0042x over previous
import functools

import jax
import jax.numpy as jnp
from jax import lax
from jax.experimental import pallas as pl
from jax.experimental.pallas import tpu as pltpu
from jax.experimental.pallas import tpu_sc as plsc

N = 10000
E = 160000
D = 256
ED = 16
H = 2 * D

NPAD = 10240
NC = 2
NS = 16
CHUNK = 80
SPAN = E // NS
NCHUNK = SPAN // CHUNK
ROWS_PER_TILE = NPAD // NS
CH = CHUNK
NCH = NCHUNK
RPT = ROWS_PER_TILE
CROWS0 = 63


def _ea_stats_body(ea_ref, s_ref, ss_ref):
    @pl.when(pl.program_id(0) == 0)
    def _():
        s_ref[...] = jnp.zeros_like(s_ref)
        ss_ref[...] = jnp.zeros_like(ss_ref)
    ea = ea_ref[...]
    s_ref[...] += jnp.sum(ea, axis=0, keepdims=True)
    ss_ref[...] += jnp.sum(ea * ea, axis=0, keepdims=True)


def _ea_stats(edge_attr):
    be = 8000
    return pl.pallas_call(
        _ea_stats_body,
        grid=(E // be,),
        in_specs=[pl.BlockSpec((be, ED), lambda i: (i, 0))],
        out_specs=[pl.BlockSpec((1, ED), lambda i: (0, 0))] * 2,
        out_shape=[jax.ShapeDtypeStruct((1, ED), jnp.float32)] * 2,
        compiler_params=pltpu.CompilerParams(
            dimension_semantics=("arbitrary",)),
    )(edge_attr)


def _h_stats_body(ea_ref, w1_ref, b1_ref, s_ref, ss_ref):
    @pl.when(pl.program_id(0) == 0)
    def _():
        s_ref[...] = jnp.zeros_like(s_ref)
        ss_ref[...] = jnp.zeros_like(ss_ref)
    h = jnp.dot(ea_ref[...], w1_ref[...],
                preferred_element_type=jnp.float32) + b1_ref[...]
    h = jnp.maximum(h, 0.0)
    s_ref[...] += jnp.sum(h, axis=0, keepdims=True)
    ss_ref[...] += jnp.sum(h * h, axis=0, keepdims=True)


def _h_stats(edge_attr, w1f, b1f):
    be = 2000
    return pl.pallas_call(
        _h_stats_body,
        grid=(E // be,),
        in_specs=[
            pl.BlockSpec((be, ED), lambda i: (i, 0)),
            pl.BlockSpec((ED, H), lambda i: (0, 0)),
            pl.BlockSpec((1, H), lambda i: (0, 0)),
        ],
        out_specs=[pl.BlockSpec((1, H), lambda i: (0, 0))] * 2,
        out_shape=[jax.ShapeDtypeStruct((1, H), jnp.float32)] * 2,
        compiler_params=pltpu.CompilerParams(
            dimension_semantics=("arbitrary",)),
    )(edge_attr, w1f, b1f)


def _edge_w_body(ea_ref, w1_ref, b1_ref, w2_ref, b2_ref, lo_ref, hi_ref):
    h = jnp.dot(ea_ref[...], w1_ref[...],
                preferred_element_type=jnp.float32) + b1_ref[...]
    h = jnp.maximum(h, 0.0)
    w = jnp.dot(h, w2_ref[...],
                preferred_element_type=jnp.float32) + b2_ref[...]
    lo_ref[...] = w[:, :128]
    hi_ref[...] = w[:, 128:]


def _edge_w(edge_attr, w1f, b1f, w2f, b2f):
    be = 2000
    return pl.pallas_call(
        _edge_w_body,
        grid=(E // be,),
        in_specs=[
            pl.BlockSpec((be, ED), lambda i: (i, 0)),
            pl.BlockSpec((ED, H), lambda i: (0, 0)),
            pl.BlockSpec((1, H), lambda i: (0, 0)),
            pl.BlockSpec((H, D), lambda i: (0, 0)),
            pl.BlockSpec((1, D), lambda i: (0, 0)),
        ],
        out_specs=[pl.BlockSpec((be, 128), lambda i: (i, 0))] * 2,
        out_shape=[jax.ShapeDtypeStruct((E, 128), jnp.float32)] * 2,
        compiler_params=pltpu.CompilerParams(
            dimension_semantics=("arbitrary",)),
    )(edge_attr, w1f, b1f, w2f, b2f)


def _node1_body(alo_ref, ahi_ref, ca_ref, cb_ref, bias_ref,
                m0_ref, s_ref, ss_ref):
    @pl.when(pl.program_id(0) == 0)
    def _():
        s_ref[...] = jnp.zeros_like(s_ref)
        ss_ref[...] = jnp.zeros_like(ss_ref)
    agg = jnp.concatenate([alo_ref[...], ahi_ref[...]], axis=1)
    cnt = jnp.maximum(ca_ref[:, 0:1] + cb_ref[:, 0:1], 1.0)
    m0 = jnp.maximum(agg / cnt + bias_ref[...], 0.0)
    m0_ref[...] = m0
    s_ref[...] += jnp.sum(m0, axis=0, keepdims=True)
    ss_ref[...] += jnp.sum(m0 * m0, axis=0, keepdims=True)


def _node1(agg_lo, agg_hi, cnt_a, cnt_b, conv_bias):
    bn = 2000
    return pl.pallas_call(
        _node1_body,
        grid=(N // bn,),
        in_specs=[
            pl.BlockSpec((bn, 128), lambda i: (i, 0)),
            pl.BlockSpec((bn, 128), lambda i: (i, 0)),
            pl.BlockSpec((bn, 128), lambda i: (i, 0)),
            pl.BlockSpec((bn, 128), lambda i: (i, 0)),
            pl.BlockSpec((1, D), lambda i: (0, 0)),
        ],
        out_specs=[
            pl.BlockSpec((bn, D), lambda i: (i, 0)),
            pl.BlockSpec((1, D), lambda i: (0, 0)),
            pl.BlockSpec((1, D), lambda i: (0, 0)),
        ],
        out_shape=[
            jax.ShapeDtypeStruct((N, D), jnp.float32),
            jax.ShapeDtypeStruct((1, D), jnp.float32),
            jax.ShapeDtypeStruct((1, D), jnp.float32),
        ],
        compiler_params=pltpu.CompilerParams(
            dimension_semantics=("arbitrary",)),
    )(agg_lo, agg_hi, cnt_a, cnt_b, conv_bias)


def _node2_body(m0_ref, w3_ref, b3_ref, t_ref, s_ref, ss_ref):
    @pl.when(pl.program_id(0) == 0)
    def _():
        s_ref[...] = jnp.zeros_like(s_ref)
        ss_ref[...] = jnp.zeros_like(ss_ref)
    t = jnp.dot(m0_ref[...], w3_ref[...],
                preferred_element_type=jnp.float32) + b3_ref[...]
    t = jnp.maximum(t, 0.0)
    t_ref[...] = t
    s_ref[...] += jnp.sum(t, axis=0, keepdims=True)
    ss_ref[...] += jnp.sum(t * t, axis=0, keepdims=True)


def _node2(m0, w3f, b3f):
    bn = 2000
    return pl.pallas_call(
        _node2_body,
        grid=(N // bn,),
        in_specs=[
            pl.BlockSpec((bn, D), lambda i: (i, 0)),
            pl.BlockSpec((D, H), lambda i: (0, 0)),
            pl.BlockSpec((1, H), lambda i: (0, 0)),
        ],
        out_specs=[
            pl.BlockSpec((bn, H), lambda i: (i, 0)),
            pl.BlockSpec((1, H), lambda i: (0, 0)),
            pl.BlockSpec((1, H), lambda i: (0, 0)),
        ],
        out_shape=[
            jax.ShapeDtypeStruct((N, H), jnp.float32),
            jax.ShapeDtypeStruct((1, H), jnp.float32),
            jax.ShapeDtypeStruct((1, H), jnp.float32),
        ],
        compiler_params=pltpu.CompilerParams(
            dimension_semantics=("arbitrary",)),
    )(m0, w3f, b3f)


def _node3_body(x_ref, t_ref, w4_ref, b4_ref, o_ref):
    o_ref[...] = x_ref[...] + jnp.dot(
        t_ref[...], w4_ref[...],
        preferred_element_type=jnp.float32) + b4_ref[...]


def _node3(x, t, w4f, b4f):
    bn = 2000
    return pl.pallas_call(
        _node3_body,
        grid=(N // bn,),
        in_specs=[
            pl.BlockSpec((bn, D), lambda i: (i, 0)),
            pl.BlockSpec((bn, H), lambda i: (i, 0)),
            pl.BlockSpec((H, D), lambda i: (0, 0)),
            pl.BlockSpec((1, D), lambda i: (0, 0)),
        ],
        out_specs=pl.BlockSpec((bn, D), lambda i: (i, 0)),
        out_shape=jax.ShapeDtypeStruct((N, D), jnp.float32),
        compiler_params=pltpu.CompilerParams(
            dimension_semantics=("arbitrary",)),
    )(x, t, w4f, b4f)


def _sc_body(xlo, xhi, wlo, whi, src_hbm, dst_hbm,
         agg_lo, agg_hi, cnt_a, cnt_b,
         src_a, src_b, dst_a, dst_b, w_a, w_b, xg_a, xg_b, agg_sh,
         sem_wa, sem_wb, sem_ga, sem_gb, sem_sa, sem_sb):
    c = lax.axis_index("c")
    s = lax.axis_index("s")

    SRC = {0: src_a, 1: src_b}
    DST = {0: dst_a, 1: dst_b}
    W = {0: w_a, 1: w_b}
    XG = {0: xg_a, 1: xg_b}
    SW = {0: sem_wa, 1: sem_wb}
    SG = {0: sem_ga, 1: sem_gb}
    SS = {0: sem_sa, 1: sem_sb}

    def zero_vmem(ref, rows):
        z = jnp.zeros((16,), jnp.float32)

        def row(b, _):
            for j in range(8):
                ref[b, pl.ds(j * 16, 16)] = z
            return 0
        lax.fori_loop(0, rows, row, 0)

    def zero_stripe():
        zero_vmem(xg_a, CH)
        for r in range(RPT // CH):
            pltpu.sync_copy(
                xg_a, agg_sh.at[pl.ds(s * RPT + r * CH, CH)])

    def half(x_hbm, w_hbm, agg_hbm, cnt_hbm, jlo, jhi):
        zero_stripe()
        plsc.subcore_barrier()

        def issue_loads(cid, q):
            base = s * SPAN + cid * CH
            pltpu.async_copy(src_hbm.at[pl.ds(base, CH)], SRC[q], SW[q])
            pltpu.async_copy(dst_hbm.at[pl.ds(base, CH)], DST[q], SW[q])
            pltpu.async_copy(w_hbm.at[pl.ds(base, CH)], W[q], SW[q])

        def wait_loads(q):
            pltpu.make_async_copy(src_hbm.at[pl.ds(0, CH)], SRC[q],
                                  SW[q]).wait()
            pltpu.make_async_copy(dst_hbm.at[pl.ds(0, CH)], DST[q],
                                  SW[q]).wait()
            pltpu.make_async_copy(w_hbm.at[pl.ds(0, CH)], W[q], SW[q]).wait()

        def wait_g(q):
            pltpu.make_async_copy(x_hbm.at[pl.ds(0, CH)], XG[q],
                                  SG[q]).wait()

        def wait_s(q):
            pltpu.make_async_copy(w_hbm.at[pl.ds(0, CH)], W[q], SS[q]).wait()

        def mul(q):
            wq, xq = W[q], XG[q]

            def mul_row(b, _):
                for j in range(8):
                    sl = pl.ds(j * 16, 16)
                    wq[b, sl] = wq[b, sl] * xq[b, sl]
                return 0
            lax.fori_loop(0, CH, mul_row, 0)

        def sub(cid, q, has_prev, has_next):
            p = 1 - q
            wait_loads(q)
            pltpu.async_copy(x_hbm.at[SRC[q]], XG[q], SG[q])
            if has_prev is True:
                wait_s(p)
            else:
                @pl.when(has_prev)
                def _():
                    wait_s(p)
            if has_next:
                issue_loads(cid + 1, p)
            wait_g(q)
            mul(q)
            pltpu.async_copy(W[q], agg_sh.at[DST[q]], SS[q], add=True)

        issue_loads(0, 0)

        def pair(g, _):
            sub(2 * g, 0, g > 0, True)
            sub(2 * g + 1, 1, True, True)
            return 0

        lax.fori_loop(0, NCH // 2, pair, 0)
        sub(NCH - 1, 0, True, False)
        wait_s(0)
        plsc.subcore_barrier()
        row0 = s * RPT
        pltpu.sync_copy(agg_sh.at[pl.ds(row0, RPT)],
                        agg_hbm.at[pl.ds(row0, RPT)])

        zero_stripe()
        zero_vmem(w_a, CH)
        e0 = jnp.where(lax.iota(jnp.int32, 16) == 0, 1.0, 0.0)

        def srow(b, _):
            w_a[b, pl.ds(0, 16)] = e0
            return 0
        lax.fori_loop(0, CH, srow, 0)
        plsc.subcore_barrier()

        def cissue(j, q):
            base = s * SPAN + j * CH
            pltpu.async_copy(dst_hbm.at[pl.ds(base, CH)], DST[q], SW[q])

        def cwait(q):
            pltpu.make_async_copy(dst_hbm.at[pl.ds(0, CH)], DST[q],
                                  SW[q]).wait()

        def csub(j, q, has_prev, has_next):
            p = 1 - q
            cwait(q)
            if has_prev is True:
                wait_s(p)
            elif has_prev is not False:
                @pl.when(has_prev)
                def _():
                    wait_s(p)
            if has_next:
                cissue(j + 1, p)
            pltpu.async_copy(w_a, agg_sh.at[DST[q]], SS[q], add=True)

        ncnt = jhi - jlo
        npair = ncnt // 2
        odd = ncnt % 2 == 1
        cissue(jlo, 0)

        def cpair(g, _):
            j = jlo + 2 * g
            csub(j, 0, g > 0, True)
            csub(j + 1, 1, True, True)
            return 0

        lax.fori_loop(0, npair - 1, cpair, 0)
        j = jlo + 2 * (npair - 1)
        csub(j, 0, npair > 1, True)
        csub(j + 1, 1, True, odd)
        if odd:
            csub(jhi - 1, 0, True, False)
            wait_s(0)
        else:
            wait_s(1)
        plsc.subcore_barrier()
        pltpu.sync_copy(agg_sh.at[pl.ds(row0, RPT)],
                        cnt_hbm.at[pl.ds(row0, RPT)])

    @pl.when(c == 0)
    def _():
        half(xlo, wlo, agg_lo, cnt_a, 0, CROWS0)

    @pl.when(c == 1)
    def _():
        half(xhi, whi, agg_hi, cnt_b, CROWS0, NCH)


@functools.cache
def _sc_aggregate():
    return pl.kernel(
        _sc_body,
        out_type=[jax.ShapeDtypeStruct((NPAD, 128), jnp.float32)] * 4,
        mesh=plsc.VectorSubcoreMesh(
            core_axis_name="c", subcore_axis_name="s",
            num_cores=NC, num_subcores=NS),
        scratch_types=[pltpu.VMEM((CH,), jnp.int32),
                       pltpu.VMEM((CH,), jnp.int32),
                       pltpu.VMEM((CH,), jnp.int32),
                       pltpu.VMEM((CH,), jnp.int32),
                       pltpu.VMEM((CH, 128), jnp.float32),
                       pltpu.VMEM((CH, 128), jnp.float32),
                       pltpu.VMEM((CH, 128), jnp.float32),
                       pltpu.VMEM((CH, 128), jnp.float32),
                       pltpu.VMEM_SHARED((NPAD, 128), jnp.float32),
                       pltpu.SemaphoreType.DMA,
                       pltpu.SemaphoreType.DMA,
                       pltpu.SemaphoreType.DMA,
                       pltpu.SemaphoreType.DMA,
                       pltpu.SemaphoreType.DMA,
                       pltpu.SemaphoreType.DMA])


def kernel(x, edge_index, edge_attr, g1, b1, W1, bw1, g2, b2, W2, bw2,
           conv_bias, g3, b3, W3, bw3, g4, b4, W4, bw4):
    src = edge_index[0]
    dst = edge_index[1]
    x_lo = x[:, :128]
    x_hi = x[:, 128:]

    s1, ss1 = _ea_stats(edge_attr)
    mu1 = s1[0] / E
    var1 = ss1[0] / E - mu1 * mu1
    sc1 = g1 / jnp.sqrt(var1 + 1e-5)
    w1f = W1 * sc1[:, None]
    b1f = (bw1 + (b1 - mu1 * sc1) @ W1)[None, :]

    s2, ss2 = _h_stats(edge_attr, w1f, b1f)
    mu2 = s2[0] / E
    var2 = ss2[0] / E - mu2 * mu2
    sc2 = g2 / jnp.sqrt(var2 + 1e-5)
    w2f = W2 * sc2[:, None]
    b2f = (bw2 + (b2 - mu2 * sc2) @ W2)[None, :]

    w_lo, w_hi = _edge_w(edge_attr, w1f, b1f, w2f, b2f)

    agg_lo, agg_hi, cnt_a, cnt_b = _sc_aggregate()(
        x_lo, x_hi, w_lo, w_hi, src, dst)

    m0, s3, ss3 = _node1(agg_lo, agg_hi, cnt_a, cnt_b, conv_bias[None, :])
    mu3 = s3[0] / N
    var3 = ss3[0] / N - mu3 * mu3
    sc3 = g3 / jnp.sqrt(var3 + 1e-5)
    w3f = W3 * sc3[:, None]
    b3f = (bw3 + (b3 - mu3 * sc3) @ W3)[None, :]

    t, s4, ss4 = _node2(m0, w3f, b3f)
    mu4 = s4[0] / N
    var4 = ss4[0] / N - mu4 * mu4
    sc4 = g4 / jnp.sqrt(var4 + 1e-5)
    w4f = W4 * sc4[:, None]
    b4f = (bw4 + (b4 - mu4 * sc4) @ W4)[None, :]

    return _node3(x, t, w4f, b4f)


# speedup vs baseline: 1.0783x; 1.0738x over previous
import functools

import jax
import jax.numpy as jnp
from jax import lax
from jax.experimental import pallas as pl
from jax.experimental.pallas import tpu as pltpu
from jax.experimental.pallas import tpu_sc as plsc

N = 10000
E = 160000
D = 256
ED = 16
H = 2 * D

NPAD = 10240
NC = 2
NS = 16
CHUNK = 80
SPAN = E // NS
NCHUNK = SPAN // CHUNK
ROWS_PER_TILE = NPAD // NS
CH = CHUNK
NCH = NCHUNK
RPT = ROWS_PER_TILE
CROWS0 = 63


def _ea_stats_body(ea_ref, s_ref, ss_ref):
    @pl.when(pl.program_id(0) == 0)
    def _():
        s_ref[...] = jnp.zeros_like(s_ref)
        ss_ref[...] = jnp.zeros_like(ss_ref)
    ea = ea_ref[...]
    s_ref[...] += jnp.sum(ea, axis=0, keepdims=True)
    ss_ref[...] += jnp.sum(ea * ea, axis=0, keepdims=True)


def _ea_stats(edge_attr):
    be = 8000
    return pl.pallas_call(
        _ea_stats_body,
        grid=(E // be,),
        in_specs=[pl.BlockSpec((be, ED), lambda i: (i, 0))],
        out_specs=[pl.BlockSpec((1, ED), lambda i: (0, 0))] * 2,
        out_shape=[jax.ShapeDtypeStruct((1, ED), jnp.float32)] * 2,
        compiler_params=pltpu.CompilerParams(
            dimension_semantics=("arbitrary",)),
    )(edge_attr)


def _h_stats_body(ea_ref, w1_ref, b1_ref, s_ref, ss_ref):
    @pl.when(pl.program_id(0) == 0)
    def _():
        s_ref[...] = jnp.zeros_like(s_ref)
        ss_ref[...] = jnp.zeros_like(ss_ref)
    h = jnp.dot(ea_ref[...], w1_ref[...],
                preferred_element_type=jnp.float32) + b1_ref[...]
    h = jnp.maximum(h, 0.0)
    s_ref[...] += jnp.sum(h, axis=0, keepdims=True)
    ss_ref[...] += jnp.sum(h * h, axis=0, keepdims=True)


def _h_stats(edge_attr, w1f, b1f):
    be = 4000
    return pl.pallas_call(
        _h_stats_body,
        grid=(E // be,),
        in_specs=[
            pl.BlockSpec((be, ED), lambda i: (i, 0)),
            pl.BlockSpec((ED, H), lambda i: (0, 0)),
            pl.BlockSpec((1, H), lambda i: (0, 0)),
        ],
        out_specs=[pl.BlockSpec((1, H), lambda i: (0, 0))] * 2,
        out_shape=[jax.ShapeDtypeStruct((1, H), jnp.float32)] * 2,
        compiler_params=pltpu.CompilerParams(
            dimension_semantics=("arbitrary",)),
    )(edge_attr, w1f, b1f)


def _edge_w_body(ea_ref, w1_ref, b1_ref, w2_ref, b2_ref, lo_ref, hi_ref):
    h = jnp.dot(ea_ref[...], w1_ref[...],
                preferred_element_type=jnp.float32) + b1_ref[...]
    h = jnp.maximum(h, 0.0)
    w = jnp.dot(h, w2_ref[...],
                preferred_element_type=jnp.float32) + b2_ref[...]
    lo_ref[...] = w[:, :128]
    hi_ref[...] = w[:, 128:]


def _edge_w(edge_attr, w1f, b1f, w2f, b2f):
    be = 4000
    return pl.pallas_call(
        _edge_w_body,
        grid=(E // be,),
        in_specs=[
            pl.BlockSpec((be, ED), lambda i: (i, 0)),
            pl.BlockSpec((ED, H), lambda i: (0, 0)),
            pl.BlockSpec((1, H), lambda i: (0, 0)),
            pl.BlockSpec((H, D), lambda i: (0, 0)),
            pl.BlockSpec((1, D), lambda i: (0, 0)),
        ],
        out_specs=[pl.BlockSpec((be, 128), lambda i: (i, 0))] * 2,
        out_shape=[jax.ShapeDtypeStruct((E, 128), jnp.float32)] * 2,
        compiler_params=pltpu.CompilerParams(
            dimension_semantics=("arbitrary",)),
    )(edge_attr, w1f, b1f, w2f, b2f)


def _node1_body(alo_ref, ahi_ref, ca_ref, cb_ref, bias_ref,
                m0_ref, s_ref, ss_ref):
    @pl.when(pl.program_id(0) == 0)
    def _():
        s_ref[...] = jnp.zeros_like(s_ref)
        ss_ref[...] = jnp.zeros_like(ss_ref)
    agg = jnp.concatenate([alo_ref[...], ahi_ref[...]], axis=1)
    cnt = jnp.maximum(ca_ref[:, 0:1] + cb_ref[:, 0:1], 1.0)
    m0 = jnp.maximum(agg / cnt + bias_ref[...], 0.0)
    m0_ref[...] = m0
    s_ref[...] += jnp.sum(m0, axis=0, keepdims=True)
    ss_ref[...] += jnp.sum(m0 * m0, axis=0, keepdims=True)


def _node1(agg_lo, agg_hi, cnt_a, cnt_b, conv_bias):
    bn = 2000
    return pl.pallas_call(
        _node1_body,
        grid=(N // bn,),
        in_specs=[
            pl.BlockSpec((bn, 128), lambda i: (i, 0)),
            pl.BlockSpec((bn, 128), lambda i: (i, 0)),
            pl.BlockSpec((bn, 128), lambda i: (i, 0)),
            pl.BlockSpec((bn, 128), lambda i: (i, 0)),
            pl.BlockSpec((1, D), lambda i: (0, 0)),
        ],
        out_specs=[
            pl.BlockSpec((bn, D), lambda i: (i, 0)),
            pl.BlockSpec((1, D), lambda i: (0, 0)),
            pl.BlockSpec((1, D), lambda i: (0, 0)),
        ],
        out_shape=[
            jax.ShapeDtypeStruct((N, D), jnp.float32),
            jax.ShapeDtypeStruct((1, D), jnp.float32),
            jax.ShapeDtypeStruct((1, D), jnp.float32),
        ],
        compiler_params=pltpu.CompilerParams(
            dimension_semantics=("arbitrary",)),
    )(agg_lo, agg_hi, cnt_a, cnt_b, conv_bias)


def _node2_body(m0_ref, w3_ref, b3_ref, t_ref, s_ref, ss_ref):
    @pl.when(pl.program_id(0) == 0)
    def _():
        s_ref[...] = jnp.zeros_like(s_ref)
        ss_ref[...] = jnp.zeros_like(ss_ref)
    t = jnp.dot(m0_ref[...], w3_ref[...],
                preferred_element_type=jnp.float32) + b3_ref[...]
    t = jnp.maximum(t, 0.0)
    t_ref[...] = t
    s_ref[...] += jnp.sum(t, axis=0, keepdims=True)
    ss_ref[...] += jnp.sum(t * t, axis=0, keepdims=True)


def _node2(m0, w3f, b3f):
    bn = 2000
    return pl.pallas_call(
        _node2_body,
        grid=(N // bn,),
        in_specs=[
            pl.BlockSpec((bn, D), lambda i: (i, 0)),
            pl.BlockSpec((D, H), lambda i: (0, 0)),
            pl.BlockSpec((1, H), lambda i: (0, 0)),
        ],
        out_specs=[
            pl.BlockSpec((bn, H), lambda i: (i, 0)),
            pl.BlockSpec((1, H), lambda i: (0, 0)),
            pl.BlockSpec((1, H), lambda i: (0, 0)),
        ],
        out_shape=[
            jax.ShapeDtypeStruct((N, H), jnp.float32),
            jax.ShapeDtypeStruct((1, H), jnp.float32),
            jax.ShapeDtypeStruct((1, H), jnp.float32),
        ],
        compiler_params=pltpu.CompilerParams(
            dimension_semantics=("arbitrary",)),
    )(m0, w3f, b3f)


def _node3_body(x_ref, t_ref, w4_ref, b4_ref, o_ref):
    o_ref[...] = x_ref[...] + jnp.dot(
        t_ref[...], w4_ref[...],
        preferred_element_type=jnp.float32) + b4_ref[...]


def _node3(x, t, w4f, b4f):
    bn = 2000
    return pl.pallas_call(
        _node3_body,
        grid=(N // bn,),
        in_specs=[
            pl.BlockSpec((bn, D), lambda i: (i, 0)),
            pl.BlockSpec((bn, H), lambda i: (i, 0)),
            pl.BlockSpec((H, D), lambda i: (0, 0)),
            pl.BlockSpec((1, D), lambda i: (0, 0)),
        ],
        out_specs=pl.BlockSpec((bn, D), lambda i: (i, 0)),
        out_shape=jax.ShapeDtypeStruct((N, D), jnp.float32),
        compiler_params=pltpu.CompilerParams(
            dimension_semantics=("arbitrary",)),
    )(x, t, w4f, b4f)


def _sc_body(xlo, xhi, wlo, whi, src_hbm, dst_hbm,
         agg_lo, agg_hi, cnt_a, cnt_b,
         src_a, src_b, dst_a, dst_b, w_a, w_b, xg_a, xg_b, agg_sh,
         sem_wa, sem_wb, sem_ga, sem_gb, sem_sa, sem_sb):
    c = lax.axis_index("c")
    s = lax.axis_index("s")

    SRC = {0: src_a, 1: src_b}
    DST = {0: dst_a, 1: dst_b}
    W = {0: w_a, 1: w_b}
    XG = {0: xg_a, 1: xg_b}
    SW = {0: sem_wa, 1: sem_wb}
    SG = {0: sem_ga, 1: sem_gb}
    SS = {0: sem_sa, 1: sem_sb}

    def zero_vmem(ref, rows):
        z = jnp.zeros((16,), jnp.float32)

        def row(b, _):
            for j in range(8):
                ref[b, pl.ds(j * 16, 16)] = z
            return 0
        lax.fori_loop(0, rows, row, 0)

    def zero_stripe():
        zero_vmem(xg_a, CH)
        for r in range(RPT // CH):
            pltpu.sync_copy(
                xg_a, agg_sh.at[pl.ds(s * RPT + r * CH, CH)])

    def half(x_hbm, w_hbm, agg_hbm, cnt_hbm, jlo, jhi):
        zero_stripe()
        plsc.subcore_barrier()

        def issue_loads(cid, q):
            base = s * SPAN + cid * CH
            pltpu.async_copy(src_hbm.at[pl.ds(base, CH)], SRC[q], SW[q])
            pltpu.async_copy(dst_hbm.at[pl.ds(base, CH)], DST[q], SW[q])
            pltpu.async_copy(w_hbm.at[pl.ds(base, CH)], W[q], SW[q])

        def wait_loads(q):
            pltpu.make_async_copy(src_hbm.at[pl.ds(0, CH)], SRC[q],
                                  SW[q]).wait()
            pltpu.make_async_copy(dst_hbm.at[pl.ds(0, CH)], DST[q],
                                  SW[q]).wait()
            pltpu.make_async_copy(w_hbm.at[pl.ds(0, CH)], W[q], SW[q]).wait()

        def wait_g(q):
            pltpu.make_async_copy(x_hbm.at[pl.ds(0, CH)], XG[q],
                                  SG[q]).wait()

        def wait_s(q):
            pltpu.make_async_copy(w_hbm.at[pl.ds(0, CH)], W[q], SS[q]).wait()

        def mul(q):
            wq, xq = W[q], XG[q]

            def mul_row(b, _):
                for j in range(8):
                    sl = pl.ds(j * 16, 16)
                    wq[b, sl] = wq[b, sl] * xq[b, sl]
                return 0
            lax.fori_loop(0, CH, mul_row, 0)

        def sub(cid, q, has_prev, has_next):
            p = 1 - q
            wait_loads(q)
            pltpu.async_copy(x_hbm.at[SRC[q]], XG[q], SG[q])
            if has_prev is True:
                wait_s(p)
            else:
                @pl.when(has_prev)
                def _():
                    wait_s(p)
            if has_next:
                issue_loads(cid + 1, p)
            wait_g(q)
            mul(q)
            pltpu.async_copy(W[q], agg_sh.at[DST[q]], SS[q], add=True)

        issue_loads(0, 0)

        def pair(g, _):
            sub(2 * g, 0, g > 0, True)
            sub(2 * g + 1, 1, True, True)
            return 0

        lax.fori_loop(0, NCH // 2, pair, 0)
        sub(NCH - 1, 0, True, False)
        wait_s(0)
        plsc.subcore_barrier()
        row0 = s * RPT
        pltpu.sync_copy(agg_sh.at[pl.ds(row0, RPT)],
                        agg_hbm.at[pl.ds(row0, RPT)])

        zero_stripe()
        zero_vmem(w_a, CH)
        e0 = jnp.where(lax.iota(jnp.int32, 16) == 0, 1.0, 0.0)

        def srow(b, _):
            w_a[b, pl.ds(0, 16)] = e0
            return 0
        lax.fori_loop(0, CH, srow, 0)
        plsc.subcore_barrier()

        def cissue(j, q):
            base = s * SPAN + j * CH
            pltpu.async_copy(dst_hbm.at[pl.ds(base, CH)], DST[q], SW[q])

        def cwait(q):
            pltpu.make_async_copy(dst_hbm.at[pl.ds(0, CH)], DST[q],
                                  SW[q]).wait()

        def csub(j, q, has_prev, has_next):
            p = 1 - q
            cwait(q)
            if has_prev is True:
                wait_s(p)
            elif has_prev is not False:
                @pl.when(has_prev)
                def _():
                    wait_s(p)
            if has_next:
                cissue(j + 1, p)
            pltpu.async_copy(w_a, agg_sh.at[DST[q]], SS[q], add=True)

        ncnt = jhi - jlo
        npair = ncnt // 2
        odd = ncnt % 2 == 1
        cissue(jlo, 0)

        def cpair(g, _):
            j = jlo + 2 * g
            csub(j, 0, g > 0, True)
            csub(j + 1, 1, True, True)
            return 0

        lax.fori_loop(0, npair - 1, cpair, 0)
        j = jlo + 2 * (npair - 1)
        csub(j, 0, npair > 1, True)
        csub(j + 1, 1, True, odd)
        if odd:
            csub(jhi - 1, 0, True, False)
            wait_s(0)
        else:
            wait_s(1)
        plsc.subcore_barrier()
        pltpu.sync_copy(agg_sh.at[pl.ds(row0, RPT)],
                        cnt_hbm.at[pl.ds(row0, RPT)])

    @pl.when(c == 0)
    def _():
        half(xlo, wlo, agg_lo, cnt_a, 0, CROWS0)

    @pl.when(c == 1)
    def _():
        half(xhi, whi, agg_hi, cnt_b, CROWS0, NCH)


@functools.cache
def _sc_aggregate():
    return pl.kernel(
        _sc_body,
        out_type=[jax.ShapeDtypeStruct((NPAD, 128), jnp.float32)] * 4,
        mesh=plsc.VectorSubcoreMesh(
            core_axis_name="c", subcore_axis_name="s",
            num_cores=NC, num_subcores=NS),
        scratch_types=[pltpu.VMEM((CH,), jnp.int32),
                       pltpu.VMEM((CH,), jnp.int32),
                       pltpu.VMEM((CH,), jnp.int32),
                       pltpu.VMEM((CH,), jnp.int32),
                       pltpu.VMEM((CH, 128), jnp.float32),
                       pltpu.VMEM((CH, 128), jnp.float32),
                       pltpu.VMEM((CH, 128), jnp.float32),
                       pltpu.VMEM((CH, 128), jnp.float32),
                       pltpu.VMEM_SHARED((NPAD, 128), jnp.float32),
                       pltpu.SemaphoreType.DMA,
                       pltpu.SemaphoreType.DMA,
                       pltpu.SemaphoreType.DMA,
                       pltpu.SemaphoreType.DMA,
                       pltpu.SemaphoreType.DMA,
                       pltpu.SemaphoreType.DMA])


def kernel(x, edge_index, edge_attr, g1, b1, W1, bw1, g2, b2, W2, bw2,
           conv_bias, g3, b3, W3, bw3, g4, b4, W4, bw4):
    src = edge_index[0]
    dst = edge_index[1]
    x_lo = x[:, :128]
    x_hi = x[:, 128:]

    s1, ss1 = _ea_stats(edge_attr)
    mu1 = s1[0] / E
    var1 = ss1[0] / E - mu1 * mu1
    sc1 = g1 / jnp.sqrt(var1 + 1e-5)
    w1f = W1 * sc1[:, None]
    b1f = (bw1 + (b1 - mu1 * sc1) @ W1)[None, :]

    s2, ss2 = _h_stats(edge_attr, w1f, b1f)
    mu2 = s2[0] / E
    var2 = ss2[0] / E - mu2 * mu2
    sc2 = g2 / jnp.sqrt(var2 + 1e-5)
    w2f = W2 * sc2[:, None]
    b2f = (bw2 + (b2 - mu2 * sc2) @ W2)[None, :]

    w_lo, w_hi = _edge_w(edge_attr, w1f, b1f, w2f, b2f)

    agg_lo, agg_hi, cnt_a, cnt_b = _sc_aggregate()(
        x_lo, x_hi, w_lo, w_hi, src, dst)

    m0, s3, ss3 = _node1(agg_lo, agg_hi, cnt_a, cnt_b, conv_bias[None, :])
    mu3 = s3[0] / N
    var3 = ss3[0] / N - mu3 * mu3
    sc3 = g3 / jnp.sqrt(var3 + 1e-5)
    w3f = W3 * sc3[:, None]
    b3f = (bw3 + (b3 - mu3 * sc3) @ W3)[None, :]

    t, s4, ss4 = _node2(m0, w3f, b3f)
    mu4 = s4[0] / N
    var4 = ss4[0] / N - mu4 * mu4
    sc4 = g4 / jnp.sqrt(var4 + 1e-5)
    w4f = W4 * sc4[:, None]
    b4f = (bw4 + (b4 - mu4 * sc4) @ W4)[None, :]

    return _node3(x, t, w4f, b4f)


# speedup vs baseline: 1.1055x; 1.0252x over previous
import functools

import jax
import jax.numpy as jnp
from jax import lax
from jax.experimental import pallas as pl
from jax.experimental.pallas import tpu as pltpu
from jax.experimental.pallas import tpu_sc as plsc

N = 10000
E = 160000
D = 256
ED = 16
H = 2 * D

NPAD = 10240
NC = 2
NS = 16
CHUNK = 80
SPAN = E // NS
NCHUNK = SPAN // CHUNK
ROWS_PER_TILE = NPAD // NS
CH = CHUNK
NCH = NCHUNK
RPT = ROWS_PER_TILE
CROWS0 = 63


def _ea_stats_body(ea_ref, s_ref, ss_ref):
    @pl.when(pl.program_id(0) == 0)
    def _():
        s_ref[...] = jnp.zeros_like(s_ref)
        ss_ref[...] = jnp.zeros_like(ss_ref)
    ea = ea_ref[...]
    s_ref[...] += jnp.sum(ea, axis=0, keepdims=True)
    ss_ref[...] += jnp.sum(ea * ea, axis=0, keepdims=True)


def _ea_stats(edge_attr):
    be = 8000
    return pl.pallas_call(
        _ea_stats_body,
        grid=(E // be,),
        in_specs=[pl.BlockSpec((be, ED), lambda i: (i, 0))],
        out_specs=[pl.BlockSpec((1, ED), lambda i: (0, 0))] * 2,
        out_shape=[jax.ShapeDtypeStruct((1, ED), jnp.float32)] * 2,
        compiler_params=pltpu.CompilerParams(
            dimension_semantics=("arbitrary",)),
    )(edge_attr)


def _h_stats_body(ea_ref, w1_ref, b1_ref, s_ref, ss_ref):
    @pl.when(pl.program_id(0) == 0)
    def _():
        s_ref[...] = jnp.zeros_like(s_ref)
        ss_ref[...] = jnp.zeros_like(ss_ref)
    h = jnp.dot(ea_ref[...], w1_ref[...],
                preferred_element_type=jnp.float32) + b1_ref[...]
    h = jnp.maximum(h, 0.0)
    s_ref[...] += jnp.sum(h, axis=0, keepdims=True)
    ss_ref[...] += jnp.sum(h * h, axis=0, keepdims=True)


def _h_stats(edge_attr, w1f, b1f):
    be = 8000
    return pl.pallas_call(
        _h_stats_body,
        grid=(E // be,),
        in_specs=[
            pl.BlockSpec((be, ED), lambda i: (i, 0)),
            pl.BlockSpec((ED, H), lambda i: (0, 0)),
            pl.BlockSpec((1, H), lambda i: (0, 0)),
        ],
        out_specs=[pl.BlockSpec((1, H), lambda i: (0, 0))] * 2,
        out_shape=[jax.ShapeDtypeStruct((1, H), jnp.float32)] * 2,
        compiler_params=pltpu.CompilerParams(
            dimension_semantics=("arbitrary",)),
    )(edge_attr, w1f, b1f)


def _edge_w_body(ea_ref, w1_ref, b1_ref, w2_ref, b2_ref, lo_ref, hi_ref):
    h = jnp.dot(ea_ref[...], w1_ref[...],
                preferred_element_type=jnp.float32) + b1_ref[...]
    h = jnp.maximum(h, 0.0)
    w = jnp.dot(h, w2_ref[...],
                preferred_element_type=jnp.float32) + b2_ref[...]
    lo_ref[...] = w[:, :128]
    hi_ref[...] = w[:, 128:]


def _edge_w(edge_attr, w1f, b1f, w2f, b2f):
    be = 8000
    return pl.pallas_call(
        _edge_w_body,
        grid=(E // be,),
        in_specs=[
            pl.BlockSpec((be, ED), lambda i: (i, 0)),
            pl.BlockSpec((ED, H), lambda i: (0, 0)),
            pl.BlockSpec((1, H), lambda i: (0, 0)),
            pl.BlockSpec((H, D), lambda i: (0, 0)),
            pl.BlockSpec((1, D), lambda i: (0, 0)),
        ],
        out_specs=[pl.BlockSpec((be, 128), lambda i: (i, 0))] * 2,
        out_shape=[jax.ShapeDtypeStruct((E, 128), jnp.float32)] * 2,
        compiler_params=pltpu.CompilerParams(
            dimension_semantics=("arbitrary",)),
    )(edge_attr, w1f, b1f, w2f, b2f)


def _node1_body(alo_ref, ahi_ref, ca_ref, cb_ref, bias_ref,
                m0_ref, s_ref, ss_ref):
    @pl.when(pl.program_id(0) == 0)
    def _():
        s_ref[...] = jnp.zeros_like(s_ref)
        ss_ref[...] = jnp.zeros_like(ss_ref)
    agg = jnp.concatenate([alo_ref[...], ahi_ref[...]], axis=1)
    cnt = jnp.maximum(ca_ref[:, 0:1] + cb_ref[:, 0:1], 1.0)
    m0 = jnp.maximum(agg / cnt + bias_ref[...], 0.0)
    m0_ref[...] = m0
    s_ref[...] += jnp.sum(m0, axis=0, keepdims=True)
    ss_ref[...] += jnp.sum(m0 * m0, axis=0, keepdims=True)


def _node1(agg_lo, agg_hi, cnt_a, cnt_b, conv_bias):
    bn = 2000
    return pl.pallas_call(
        _node1_body,
        grid=(N // bn,),
        in_specs=[
            pl.BlockSpec((bn, 128), lambda i: (i, 0)),
            pl.BlockSpec((bn, 128), lambda i: (i, 0)),
            pl.BlockSpec((bn, 128), lambda i: (i, 0)),
            pl.BlockSpec((bn, 128), lambda i: (i, 0)),
            pl.BlockSpec((1, D), lambda i: (0, 0)),
        ],
        out_specs=[
            pl.BlockSpec((bn, D), lambda i: (i, 0)),
            pl.BlockSpec((1, D), lambda i: (0, 0)),
            pl.BlockSpec((1, D), lambda i: (0, 0)),
        ],
        out_shape=[
            jax.ShapeDtypeStruct((N, D), jnp.float32),
            jax.ShapeDtypeStruct((1, D), jnp.float32),
            jax.ShapeDtypeStruct((1, D), jnp.float32),
        ],
        compiler_params=pltpu.CompilerParams(
            dimension_semantics=("arbitrary",)),
    )(agg_lo, agg_hi, cnt_a, cnt_b, conv_bias)


def _node2_body(m0_ref, w3_ref, b3_ref, t_ref, s_ref, ss_ref):
    @pl.when(pl.program_id(0) == 0)
    def _():
        s_ref[...] = jnp.zeros_like(s_ref)
        ss_ref[...] = jnp.zeros_like(ss_ref)
    t = jnp.dot(m0_ref[...], w3_ref[...],
                preferred_element_type=jnp.float32) + b3_ref[...]
    t = jnp.maximum(t, 0.0)
    t_ref[...] = t
    s_ref[...] += jnp.sum(t, axis=0, keepdims=True)
    ss_ref[...] += jnp.sum(t * t, axis=0, keepdims=True)


def _node2(m0, w3f, b3f):
    bn = 2000
    return pl.pallas_call(
        _node2_body,
        grid=(N // bn,),
        in_specs=[
            pl.BlockSpec((bn, D), lambda i: (i, 0)),
            pl.BlockSpec((D, H), lambda i: (0, 0)),
            pl.BlockSpec((1, H), lambda i: (0, 0)),
        ],
        out_specs=[
            pl.BlockSpec((bn, H), lambda i: (i, 0)),
            pl.BlockSpec((1, H), lambda i: (0, 0)),
            pl.BlockSpec((1, H), lambda i: (0, 0)),
        ],
        out_shape=[
            jax.ShapeDtypeStruct((N, H), jnp.float32),
            jax.ShapeDtypeStruct((1, H), jnp.float32),
            jax.ShapeDtypeStruct((1, H), jnp.float32),
        ],
        compiler_params=pltpu.CompilerParams(
            dimension_semantics=("arbitrary",)),
    )(m0, w3f, b3f)


def _node3_body(x_ref, t_ref, w4_ref, b4_ref, o_ref):
    o_ref[...] = x_ref[...] + jnp.dot(
        t_ref[...], w4_ref[...],
        preferred_element_type=jnp.float32) + b4_ref[...]


def _node3(x, t, w4f, b4f):
    bn = 2000
    return pl.pallas_call(
        _node3_body,
        grid=(N // bn,),
        in_specs=[
            pl.BlockSpec((bn, D), lambda i: (i, 0)),
            pl.BlockSpec((bn, H), lambda i: (i, 0)),
            pl.BlockSpec((H, D), lambda i: (0, 0)),
            pl.BlockSpec((1, D), lambda i: (0, 0)),
        ],
        out_specs=pl.BlockSpec((bn, D), lambda i: (i, 0)),
        out_shape=jax.ShapeDtypeStruct((N, D), jnp.float32),
        compiler_params=pltpu.CompilerParams(
            dimension_semantics=("arbitrary",)),
    )(x, t, w4f, b4f)


def _sc_body(xlo, xhi, wlo, whi, src_hbm, dst_hbm,
         agg_lo, agg_hi, cnt_a, cnt_b,
         src_a, src_b, dst_a, dst_b, w_a, w_b, xg_a, xg_b, agg_sh,
         sem_wa, sem_wb, sem_ga, sem_gb, sem_sa, sem_sb):
    c = lax.axis_index("c")
    s = lax.axis_index("s")

    SRC = {0: src_a, 1: src_b}
    DST = {0: dst_a, 1: dst_b}
    W = {0: w_a, 1: w_b}
    XG = {0: xg_a, 1: xg_b}
    SW = {0: sem_wa, 1: sem_wb}
    SG = {0: sem_ga, 1: sem_gb}
    SS = {0: sem_sa, 1: sem_sb}

    def zero_vmem(ref, rows):
        z = jnp.zeros((16,), jnp.float32)

        def row(b, _):
            for j in range(8):
                ref[b, pl.ds(j * 16, 16)] = z
            return 0
        lax.fori_loop(0, rows, row, 0)

    def zero_stripe():
        zero_vmem(xg_a, CH)
        for r in range(RPT // CH):
            pltpu.sync_copy(
                xg_a, agg_sh.at[pl.ds(s * RPT + r * CH, CH)])

    def half(x_hbm, w_hbm, agg_hbm, cnt_hbm, jlo, jhi):
        zero_stripe()
        plsc.subcore_barrier()

        def issue_loads(cid, q):
            base = s * SPAN + cid * CH
            pltpu.async_copy(src_hbm.at[pl.ds(base, CH)], SRC[q], SW[q])
            pltpu.async_copy(dst_hbm.at[pl.ds(base, CH)], DST[q], SW[q])
            pltpu.async_copy(w_hbm.at[pl.ds(base, CH)], W[q], SW[q])

        def wait_loads(q):
            pltpu.make_async_copy(src_hbm.at[pl.ds(0, CH)], SRC[q],
                                  SW[q]).wait()
            pltpu.make_async_copy(dst_hbm.at[pl.ds(0, CH)], DST[q],
                                  SW[q]).wait()
            pltpu.make_async_copy(w_hbm.at[pl.ds(0, CH)], W[q], SW[q]).wait()

        def wait_g(q):
            pltpu.make_async_copy(x_hbm.at[pl.ds(0, CH)], XG[q],
                                  SG[q]).wait()

        def wait_s(q):
            pltpu.make_async_copy(w_hbm.at[pl.ds(0, CH)], W[q], SS[q]).wait()

        def mul(q):
            wq, xq = W[q], XG[q]

            def mul_row(b, _):
                for j in range(8):
                    sl = pl.ds(j * 16, 16)
                    wq[b, sl] = wq[b, sl] * xq[b, sl]
                return 0
            lax.fori_loop(0, CH, mul_row, 0)

        def sub(cid, q, has_prev, has_next):
            p = 1 - q
            wait_loads(q)
            pltpu.async_copy(x_hbm.at[SRC[q]], XG[q], SG[q])
            if has_prev is True:
                wait_s(p)
            else:
                @pl.when(has_prev)
                def _():
                    wait_s(p)
            if has_next:
                issue_loads(cid + 1, p)
            wait_g(q)
            mul(q)
            pltpu.async_copy(W[q], agg_sh.at[DST[q]], SS[q], add=True)

        issue_loads(0, 0)

        def pair(g, _):
            sub(2 * g, 0, g > 0, True)
            sub(2 * g + 1, 1, True, True)
            return 0

        lax.fori_loop(0, NCH // 2, pair, 0)
        sub(NCH - 1, 0, True, False)
        wait_s(0)
        plsc.subcore_barrier()
        row0 = s * RPT
        pltpu.sync_copy(agg_sh.at[pl.ds(row0, RPT)],
                        agg_hbm.at[pl.ds(row0, RPT)])

        zero_stripe()
        zero_vmem(w_a, CH)
        e0 = jnp.where(lax.iota(jnp.int32, 16) == 0, 1.0, 0.0)

        def srow(b, _):
            w_a[b, pl.ds(0, 16)] = e0
            return 0
        lax.fori_loop(0, CH, srow, 0)
        plsc.subcore_barrier()

        def cissue(j, q):
            base = s * SPAN + j * CH
            pltpu.async_copy(dst_hbm.at[pl.ds(base, CH)], DST[q], SW[q])

        def cwait(q):
            pltpu.make_async_copy(dst_hbm.at[pl.ds(0, CH)], DST[q],
                                  SW[q]).wait()

        def csub(j, q, has_prev, has_next):
            p = 1 - q
            cwait(q)
            if has_prev is True:
                wait_s(p)
            elif has_prev is not False:
                @pl.when(has_prev)
                def _():
                    wait_s(p)
            if has_next:
                cissue(j + 1, p)
            pltpu.async_copy(w_a, agg_sh.at[DST[q]], SS[q], add=True)

        ncnt = jhi - jlo
        npair = ncnt // 2
        odd = ncnt % 2 == 1
        cissue(jlo, 0)

        def cpair(g, _):
            j = jlo + 2 * g
            csub(j, 0, g > 0, True)
            csub(j + 1, 1, True, True)
            return 0

        lax.fori_loop(0, npair - 1, cpair, 0)
        j = jlo + 2 * (npair - 1)
        csub(j, 0, npair > 1, True)
        csub(j + 1, 1, True, odd)
        if odd:
            csub(jhi - 1, 0, True, False)
            wait_s(0)
        else:
            wait_s(1)
        plsc.subcore_barrier()
        pltpu.sync_copy(agg_sh.at[pl.ds(row0, RPT)],
                        cnt_hbm.at[pl.ds(row0, RPT)])

    @pl.when(c == 0)
    def _():
        half(xlo, wlo, agg_lo, cnt_a, 0, CROWS0)

    @pl.when(c == 1)
    def _():
        half(xhi, whi, agg_hi, cnt_b, CROWS0, NCH)


@functools.cache
def _sc_aggregate():
    return pl.kernel(
        _sc_body,
        out_type=[jax.ShapeDtypeStruct((NPAD, 128), jnp.float32)] * 4,
        mesh=plsc.VectorSubcoreMesh(
            core_axis_name="c", subcore_axis_name="s",
            num_cores=NC, num_subcores=NS),
        scratch_types=[pltpu.VMEM((CH,), jnp.int32),
                       pltpu.VMEM((CH,), jnp.int32),
                       pltpu.VMEM((CH,), jnp.int32),
                       pltpu.VMEM((CH,), jnp.int32),
                       pltpu.VMEM((CH, 128), jnp.float32),
                       pltpu.VMEM((CH, 128), jnp.float32),
                       pltpu.VMEM((CH, 128), jnp.float32),
                       pltpu.VMEM((CH, 128), jnp.float32),
                       pltpu.VMEM_SHARED((NPAD, 128), jnp.float32),
                       pltpu.SemaphoreType.DMA,
                       pltpu.SemaphoreType.DMA,
                       pltpu.SemaphoreType.DMA,
                       pltpu.SemaphoreType.DMA,
                       pltpu.SemaphoreType.DMA,
                       pltpu.SemaphoreType.DMA])


def kernel(x, edge_index, edge_attr, g1, b1, W1, bw1, g2, b2, W2, bw2,
           conv_bias, g3, b3, W3, bw3, g4, b4, W4, bw4):
    src = edge_index[0]
    dst = edge_index[1]
    x_lo = x[:, :128]
    x_hi = x[:, 128:]

    s1, ss1 = _ea_stats(edge_attr)
    mu1 = s1[0] / E
    var1 = ss1[0] / E - mu1 * mu1
    sc1 = g1 / jnp.sqrt(var1 + 1e-5)
    w1f = W1 * sc1[:, None]
    b1f = (bw1 + (b1 - mu1 * sc1) @ W1)[None, :]

    s2, ss2 = _h_stats(edge_attr, w1f, b1f)
    mu2 = s2[0] / E
    var2 = ss2[0] / E - mu2 * mu2
    sc2 = g2 / jnp.sqrt(var2 + 1e-5)
    w2f = W2 * sc2[:, None]
    b2f = (bw2 + (b2 - mu2 * sc2) @ W2)[None, :]

    w_lo, w_hi = _edge_w(edge_attr, w1f, b1f, w2f, b2f)

    agg_lo, agg_hi, cnt_a, cnt_b = _sc_aggregate()(
        x_lo, x_hi, w_lo, w_hi, src, dst)

    m0, s3, ss3 = _node1(agg_lo, agg_hi, cnt_a, cnt_b, conv_bias[None, :])
    mu3 = s3[0] / N
    var3 = ss3[0] / N - mu3 * mu3
    sc3 = g3 / jnp.sqrt(var3 + 1e-5)
    w3f = W3 * sc3[:, None]
    b3f = (bw3 + (b3 - mu3 * sc3) @ W3)[None, :]

    t, s4, ss4 = _node2(m0, w3f, b3f)
    mu4 = s4[0] / N
    var4 = ss4[0] / N - mu4 * mu4
    sc4 = g4 / jnp.sqrt(var4 + 1e-5)
    w4f = W4 * sc4[:, None]
    b4f = (bw4 + (b4 - mu4 * sc4) @ W4)[None, :]

    return _node3(x, t, w4f, b4f)


# speedup vs baseline: 1.1090x; 1.0031x over previous
import functools

import jax
import jax.numpy as jnp
from jax import lax
from jax.experimental import pallas as pl
from jax.experimental.pallas import tpu as pltpu
from jax.experimental.pallas import tpu_sc as plsc

N = 10000
E = 160000
D = 256
ED = 16
H = 2 * D

NPAD = 10240
NC = 2
NS = 16
CHUNK = 80
SPAN = E // NS
NCHUNK = SPAN // CHUNK
ROWS_PER_TILE = NPAD // NS
CH = CHUNK
NCH = NCHUNK
RPT = ROWS_PER_TILE
CROWS0 = 63


def _ea_stats_body(ea_ref, s_ref, ss_ref):
    @pl.when(pl.program_id(0) == 0)
    def _():
        s_ref[...] = jnp.zeros_like(s_ref)
        ss_ref[...] = jnp.zeros_like(ss_ref)
    ea = ea_ref[...]
    s_ref[...] += jnp.sum(ea, axis=0, keepdims=True)
    ss_ref[...] += jnp.sum(ea * ea, axis=0, keepdims=True)


def _ea_stats(edge_attr):
    be = 8000
    return pl.pallas_call(
        _ea_stats_body,
        grid=(E // be,),
        in_specs=[pl.BlockSpec((be, ED), lambda i: (i, 0))],
        out_specs=[pl.BlockSpec((1, ED), lambda i: (0, 0))] * 2,
        out_shape=[jax.ShapeDtypeStruct((1, ED), jnp.float32)] * 2,
        compiler_params=pltpu.CompilerParams(
            dimension_semantics=("arbitrary",)),
    )(edge_attr)


def _h_stats_body(ea_ref, w1_ref, b1_ref, s_ref, ss_ref):
    @pl.when(pl.program_id(0) == 0)
    def _():
        s_ref[...] = jnp.zeros_like(s_ref)
        ss_ref[...] = jnp.zeros_like(ss_ref)
    h = jnp.dot(ea_ref[...], w1_ref[...],
                preferred_element_type=jnp.float32) + b1_ref[...]
    h = jnp.maximum(h, 0.0)
    s_ref[...] += jnp.sum(h, axis=0, keepdims=True)
    ss_ref[...] += jnp.sum(h * h, axis=0, keepdims=True)


def _h_stats(edge_attr, w1f, b1f):
    be = 8000
    return pl.pallas_call(
        _h_stats_body,
        grid=(E // be,),
        in_specs=[
            pl.BlockSpec((be, ED), lambda i: (i, 0)),
            pl.BlockSpec((ED, H), lambda i: (0, 0)),
            pl.BlockSpec((1, H), lambda i: (0, 0)),
        ],
        out_specs=[pl.BlockSpec((1, H), lambda i: (0, 0))] * 2,
        out_shape=[jax.ShapeDtypeStruct((1, H), jnp.float32)] * 2,
        compiler_params=pltpu.CompilerParams(
            dimension_semantics=("arbitrary",)),
    )(edge_attr, w1f, b1f)


def _edge_w_body(ea_ref, w1_ref, b1_ref, w2_ref, b2_ref, lo_ref, hi_ref):
    h = jnp.dot(ea_ref[...], w1_ref[...],
                preferred_element_type=jnp.float32) + b1_ref[...]
    h = jnp.maximum(h, 0.0)
    w = jnp.dot(h, w2_ref[...],
                preferred_element_type=jnp.float32) + b2_ref[...]
    lo_ref[...] = w[:, :128]
    hi_ref[...] = w[:, 128:]


def _edge_w(edge_attr, w1f, b1f, w2f, b2f):
    be = 10000
    return pl.pallas_call(
        _edge_w_body,
        grid=(E // be,),
        in_specs=[
            pl.BlockSpec((be, ED), lambda i: (i, 0)),
            pl.BlockSpec((ED, H), lambda i: (0, 0)),
            pl.BlockSpec((1, H), lambda i: (0, 0)),
            pl.BlockSpec((H, D), lambda i: (0, 0)),
            pl.BlockSpec((1, D), lambda i: (0, 0)),
        ],
        out_specs=[pl.BlockSpec((be, 128), lambda i: (i, 0))] * 2,
        out_shape=[jax.ShapeDtypeStruct((E, 128), jnp.float32)] * 2,
        compiler_params=pltpu.CompilerParams(
            dimension_semantics=("arbitrary",)),
    )(edge_attr, w1f, b1f, w2f, b2f)


def _node1_body(alo_ref, ahi_ref, ca_ref, cb_ref, bias_ref,
                m0_ref, s_ref, ss_ref):
    @pl.when(pl.program_id(0) == 0)
    def _():
        s_ref[...] = jnp.zeros_like(s_ref)
        ss_ref[...] = jnp.zeros_like(ss_ref)
    agg = jnp.concatenate([alo_ref[...], ahi_ref[...]], axis=1)
    cnt = jnp.maximum(ca_ref[:, 0:1] + cb_ref[:, 0:1], 1.0)
    m0 = jnp.maximum(agg / cnt + bias_ref[...], 0.0)
    m0_ref[...] = m0
    s_ref[...] += jnp.sum(m0, axis=0, keepdims=True)
    ss_ref[...] += jnp.sum(m0 * m0, axis=0, keepdims=True)


def _node1(agg_lo, agg_hi, cnt_a, cnt_b, conv_bias):
    bn = 2000
    return pl.pallas_call(
        _node1_body,
        grid=(N // bn,),
        in_specs=[
            pl.BlockSpec((bn, 128), lambda i: (i, 0)),
            pl.BlockSpec((bn, 128), lambda i: (i, 0)),
            pl.BlockSpec((bn, 128), lambda i: (i, 0)),
            pl.BlockSpec((bn, 128), lambda i: (i, 0)),
            pl.BlockSpec((1, D), lambda i: (0, 0)),
        ],
        out_specs=[
            pl.BlockSpec((bn, D), lambda i: (i, 0)),
            pl.BlockSpec((1, D), lambda i: (0, 0)),
            pl.BlockSpec((1, D), lambda i: (0, 0)),
        ],
        out_shape=[
            jax.ShapeDtypeStruct((N, D), jnp.float32),
            jax.ShapeDtypeStruct((1, D), jnp.float32),
            jax.ShapeDtypeStruct((1, D), jnp.float32),
        ],
        compiler_params=pltpu.CompilerParams(
            dimension_semantics=("arbitrary",)),
    )(agg_lo, agg_hi, cnt_a, cnt_b, conv_bias)


def _node2_body(m0_ref, w3_ref, b3_ref, t_ref, s_ref, ss_ref):
    @pl.when(pl.program_id(0) == 0)
    def _():
        s_ref[...] = jnp.zeros_like(s_ref)
        ss_ref[...] = jnp.zeros_like(ss_ref)
    t = jnp.dot(m0_ref[...], w3_ref[...],
                preferred_element_type=jnp.float32) + b3_ref[...]
    t = jnp.maximum(t, 0.0)
    t_ref[...] = t
    s_ref[...] += jnp.sum(t, axis=0, keepdims=True)
    ss_ref[...] += jnp.sum(t * t, axis=0, keepdims=True)


def _node2(m0, w3f, b3f):
    bn = 2000
    return pl.pallas_call(
        _node2_body,
        grid=(N // bn,),
        in_specs=[
            pl.BlockSpec((bn, D), lambda i: (i, 0)),
            pl.BlockSpec((D, H), lambda i: (0, 0)),
            pl.BlockSpec((1, H), lambda i: (0, 0)),
        ],
        out_specs=[
            pl.BlockSpec((bn, H), lambda i: (i, 0)),
            pl.BlockSpec((1, H), lambda i: (0, 0)),
            pl.BlockSpec((1, H), lambda i: (0, 0)),
        ],
        out_shape=[
            jax.ShapeDtypeStruct((N, H), jnp.float32),
            jax.ShapeDtypeStruct((1, H), jnp.float32),
            jax.ShapeDtypeStruct((1, H), jnp.float32),
        ],
        compiler_params=pltpu.CompilerParams(
            dimension_semantics=("arbitrary",)),
    )(m0, w3f, b3f)


def _node3_body(x_ref, t_ref, w4_ref, b4_ref, o_ref):
    o_ref[...] = x_ref[...] + jnp.dot(
        t_ref[...], w4_ref[...],
        preferred_element_type=jnp.float32) + b4_ref[...]


def _node3(x, t, w4f, b4f):
    bn = 2000
    return pl.pallas_call(
        _node3_body,
        grid=(N // bn,),
        in_specs=[
            pl.BlockSpec((bn, D), lambda i: (i, 0)),
            pl.BlockSpec((bn, H), lambda i: (i, 0)),
            pl.BlockSpec((H, D), lambda i: (0, 0)),
            pl.BlockSpec((1, D), lambda i: (0, 0)),
        ],
        out_specs=pl.BlockSpec((bn, D), lambda i: (i, 0)),
        out_shape=jax.ShapeDtypeStruct((N, D), jnp.float32),
        compiler_params=pltpu.CompilerParams(
            dimension_semantics=("arbitrary",)),
    )(x, t, w4f, b4f)


def _sc_body(xlo, xhi, wlo, whi, src_hbm, dst_hbm,
         agg_lo, agg_hi, cnt_a, cnt_b,
         src_a, src_b, dst_a, dst_b, w_a, w_b, xg_a, xg_b, agg_sh,
         sem_wa, sem_wb, sem_ga, sem_gb, sem_sa, sem_sb):
    c = lax.axis_index("c")
    s = lax.axis_index("s")

    SRC = {0: src_a, 1: src_b}
    DST = {0: dst_a, 1: dst_b}
    W = {0: w_a, 1: w_b}
    XG = {0: xg_a, 1: xg_b}
    SW = {0: sem_wa, 1: sem_wb}
    SG = {0: sem_ga, 1: sem_gb}
    SS = {0: sem_sa, 1: sem_sb}

    def zero_vmem(ref, rows):
        z = jnp.zeros((16,), jnp.float32)

        def row(b, _):
            for j in range(8):
                ref[b, pl.ds(j * 16, 16)] = z
            return 0
        lax.fori_loop(0, rows, row, 0)

    def zero_stripe():
        zero_vmem(xg_a, CH)
        for r in range(RPT // CH):
            pltpu.sync_copy(
                xg_a, agg_sh.at[pl.ds(s * RPT + r * CH, CH)])

    def half(x_hbm, w_hbm, agg_hbm, cnt_hbm, jlo, jhi):
        zero_stripe()
        plsc.subcore_barrier()

        def issue_loads(cid, q):
            base = s * SPAN + cid * CH
            pltpu.async_copy(src_hbm.at[pl.ds(base, CH)], SRC[q], SW[q])
            pltpu.async_copy(dst_hbm.at[pl.ds(base, CH)], DST[q], SW[q])
            pltpu.async_copy(w_hbm.at[pl.ds(base, CH)], W[q], SW[q])

        def wait_loads(q):
            pltpu.make_async_copy(src_hbm.at[pl.ds(0, CH)], SRC[q],
                                  SW[q]).wait()
            pltpu.make_async_copy(dst_hbm.at[pl.ds(0, CH)], DST[q],
                                  SW[q]).wait()
            pltpu.make_async_copy(w_hbm.at[pl.ds(0, CH)], W[q], SW[q]).wait()

        def wait_g(q):
            pltpu.make_async_copy(x_hbm.at[pl.ds(0, CH)], XG[q],
                                  SG[q]).wait()

        def wait_s(q):
            pltpu.make_async_copy(w_hbm.at[pl.ds(0, CH)], W[q], SS[q]).wait()

        def mul(q):
            wq, xq = W[q], XG[q]

            def mul_row(b, _):
                for j in range(8):
                    sl = pl.ds(j * 16, 16)
                    wq[b, sl] = wq[b, sl] * xq[b, sl]
                return 0
            lax.fori_loop(0, CH, mul_row, 0)

        def sub(cid, q, has_prev, has_next):
            p = 1 - q
            wait_loads(q)
            pltpu.async_copy(x_hbm.at[SRC[q]], XG[q], SG[q])
            if has_prev is True:
                wait_s(p)
            else:
                @pl.when(has_prev)
                def _():
                    wait_s(p)
            if has_next:
                issue_loads(cid + 1, p)
            wait_g(q)
            mul(q)
            pltpu.async_copy(W[q], agg_sh.at[DST[q]], SS[q], add=True)

        issue_loads(0, 0)

        def pair(g, _):
            sub(2 * g, 0, g > 0, True)
            sub(2 * g + 1, 1, True, True)
            return 0

        lax.fori_loop(0, NCH // 2, pair, 0)
        sub(NCH - 1, 0, True, False)
        wait_s(0)
        plsc.subcore_barrier()
        row0 = s * RPT
        pltpu.sync_copy(agg_sh.at[pl.ds(row0, RPT)],
                        agg_hbm.at[pl.ds(row0, RPT)])

        zero_stripe()
        zero_vmem(w_a, CH)
        e0 = jnp.where(lax.iota(jnp.int32, 16) == 0, 1.0, 0.0)

        def srow(b, _):
            w_a[b, pl.ds(0, 16)] = e0
            return 0
        lax.fori_loop(0, CH, srow, 0)
        plsc.subcore_barrier()

        def cissue(j, q):
            base = s * SPAN + j * CH
            pltpu.async_copy(dst_hbm.at[pl.ds(base, CH)], DST[q], SW[q])

        def cwait(q):
            pltpu.make_async_copy(dst_hbm.at[pl.ds(0, CH)], DST[q],
                                  SW[q]).wait()

        def csub(j, q, has_prev, has_next):
            p = 1 - q
            cwait(q)
            if has_prev is True:
                wait_s(p)
            elif has_prev is not False:
                @pl.when(has_prev)
                def _():
                    wait_s(p)
            if has_next:
                cissue(j + 1, p)
            pltpu.async_copy(w_a, agg_sh.at[DST[q]], SS[q], add=True)

        ncnt = jhi - jlo
        npair = ncnt // 2
        odd = ncnt % 2 == 1
        cissue(jlo, 0)

        def cpair(g, _):
            j = jlo + 2 * g
            csub(j, 0, g > 0, True)
            csub(j + 1, 1, True, True)
            return 0

        lax.fori_loop(0, npair - 1, cpair, 0)
        j = jlo + 2 * (npair - 1)
        csub(j, 0, npair > 1, True)
        csub(j + 1, 1, True, odd)
        if odd:
            csub(jhi - 1, 0, True, False)
            wait_s(0)
        else:
            wait_s(1)
        plsc.subcore_barrier()
        pltpu.sync_copy(agg_sh.at[pl.ds(row0, RPT)],
                        cnt_hbm.at[pl.ds(row0, RPT)])

    @pl.when(c == 0)
    def _():
        half(xlo, wlo, agg_lo, cnt_a, 0, CROWS0)

    @pl.when(c == 1)
    def _():
        half(xhi, whi, agg_hi, cnt_b, CROWS0, NCH)


@functools.cache
def _sc_aggregate():
    return pl.kernel(
        _sc_body,
        out_type=[jax.ShapeDtypeStruct((NPAD, 128), jnp.float32)] * 4,
        mesh=plsc.VectorSubcoreMesh(
            core_axis_name="c", subcore_axis_name="s",
            num_cores=NC, num_subcores=NS),
        scratch_types=[pltpu.VMEM((CH,), jnp.int32),
                       pltpu.VMEM((CH,), jnp.int32),
                       pltpu.VMEM((CH,), jnp.int32),
                       pltpu.VMEM((CH,), jnp.int32),
                       pltpu.VMEM((CH, 128), jnp.float32),
                       pltpu.VMEM((CH, 128), jnp.float32),
                       pltpu.VMEM((CH, 128), jnp.float32),
                       pltpu.VMEM((CH, 128), jnp.float32),
                       pltpu.VMEM_SHARED((NPAD, 128), jnp.float32),
                       pltpu.SemaphoreType.DMA,
                       pltpu.SemaphoreType.DMA,
                       pltpu.SemaphoreType.DMA,
                       pltpu.SemaphoreType.DMA,
                       pltpu.SemaphoreType.DMA,
                       pltpu.SemaphoreType.DMA])


def kernel(x, edge_index, edge_attr, g1, b1, W1, bw1, g2, b2, W2, bw2,
           conv_bias, g3, b3, W3, bw3, g4, b4, W4, bw4):
    src = edge_index[0]
    dst = edge_index[1]
    x_lo = x[:, :128]
    x_hi = x[:, 128:]

    s1, ss1 = _ea_stats(edge_attr)
    mu1 = s1[0] / E
    var1 = ss1[0] / E - mu1 * mu1
    sc1 = g1 / jnp.sqrt(var1 + 1e-5)
    w1f = W1 * sc1[:, None]
    b1f = (bw1 + (b1 - mu1 * sc1) @ W1)[None, :]

    s2, ss2 = _h_stats(edge_attr, w1f, b1f)
    mu2 = s2[0] / E
    var2 = ss2[0] / E - mu2 * mu2
    sc2 = g2 / jnp.sqrt(var2 + 1e-5)
    w2f = W2 * sc2[:, None]
    b2f = (bw2 + (b2 - mu2 * sc2) @ W2)[None, :]

    w_lo, w_hi = _edge_w(edge_attr, w1f, b1f, w2f, b2f)

    agg_lo, agg_hi, cnt_a, cnt_b = _sc_aggregate()(
        x_lo, x_hi, w_lo, w_hi, src, dst)

    m0, s3, ss3 = _node1(agg_lo, agg_hi, cnt_a, cnt_b, conv_bias[None, :])
    mu3 = s3[0] / N
    var3 = ss3[0] / N - mu3 * mu3
    sc3 = g3 / jnp.sqrt(var3 + 1e-5)
    w3f = W3 * sc3[:, None]
    b3f = (bw3 + (b3 - mu3 * sc3) @ W3)[None, :]

    t, s4, ss4 = _node2(m0, w3f, b3f)
    mu4 = s4[0] / N
    var4 = ss4[0] / N - mu4 * mu4
    sc4 = g4 / jnp.sqrt(var4 + 1e-5)
    w4f = W4 * sc4[:, None]
    b4f = (bw4 + (b4 - mu4 * sc4) @ W4)[None, :]

    return _node3(x, t, w4f, b4f)
